# initial kernel scaffold (unmeasured)
import jax
import jax.numpy as jnp
from jax import lax
from jax.experimental import pallas as pl
from jax.experimental.pallas import tpu as pltpu

B, QLEN, H, D = 8, 8, 16, 128
SCALE = D ** -0.5
N_DEV = 4


def _flash_body(q_ref, k_ref, v_ref, o_ref, l_ref):
    q = q_ref[0, :, 0, :]
    k = k_ref[0, :, 0, :]
    v = v_ref[0, :, 0, :]
    s = lax.dot_general(
        q.astype(jnp.bfloat16),
        k.astype(jnp.bfloat16),
        (((1,), (1,)), ((), ())),
        preferred_element_type=jnp.float32,
    ) * SCALE
    p = jnp.exp(s)
    l = jnp.sum(p, axis=1, keepdims=True)
    o = lax.dot_general(
        p.astype(jnp.bfloat16),
        v.astype(jnp.bfloat16),
        (((1,), (0,)), ((), ())),
        preferred_element_type=jnp.float32,
    )
    o_ref[0, :, 0, :] = o.astype(jnp.bfloat16)
    l_ref[0, :, 0, :] = l


def _local_flash(Q, K, V):
    kv = K.shape[1]
    return pl.pallas_call(
        _flash_body,
        grid=(B, H),
        in_specs=[
            pl.BlockSpec((1, QLEN, 1, D), lambda b, h: (b, 0, h, 0)),
            pl.BlockSpec((1, kv, 1, D), lambda b, h: (b, 0, h, 0)),
            pl.BlockSpec((1, kv, 1, D), lambda b, h: (b, 0, h, 0)),
        ],
        out_specs=[
            pl.BlockSpec((1, QLEN, 1, D), lambda b, h: (b, 0, h, 0)),
            pl.BlockSpec((1, QLEN, 1, 1), lambda b, h: (b, 0, h, 0)),
        ],
        out_shape=[
            jax.ShapeDtypeStruct((B, QLEN, H, D), jnp.bfloat16),
            jax.ShapeDtypeStruct((B, QLEN, H, 1), jnp.float32),
        ],
    )(Q, K, V)


def _combine_body(o_ref, l_ref, out_ref, co, cl, o_send, o_recv, l_send, l_recv):
    my = lax.axis_index("i")

    def exchange(o_src, l_src, partner, slot, sem):
        rdma_o = pltpu.make_async_remote_copy(
            src_ref=o_src,
            dst_ref=co.at[slot],
            send_sem=o_send.at[sem],
            recv_sem=o_recv.at[sem],
            device_id=(partner,),
            device_id_type=pl.DeviceIdType.MESH,
        )
        rdma_l = pltpu.make_async_remote_copy(
            src_ref=l_src,
            dst_ref=cl.at[slot],
            send_sem=l_send.at[sem],
            recv_sem=l_recv.at[sem],
            device_id=(partner,),
            device_id_type=pl.DeviceIdType.MESH,
        )
        rdma_o.start()
        rdma_l.start()
        rdma_o.wait()
        rdma_l.wait()

    exchange(o_ref, l_ref, my ^ 1, 0, 0)
    co[1] = (o_ref[...].astype(jnp.float32) + co[0].astype(jnp.float32)).astype(
        jnp.bfloat16
    )
    cl[1] = l_ref[...] + cl[0]

    exchange(co.at[1], cl.at[1], 3 - my, 2, 1)
    o_tot = co[1].astype(jnp.float32) + co[2].astype(jnp.float32)
    l_tot = cl[1] + cl[2]
    out_ref[...] = o_tot / l_tot


def _combine(o_part, l_part):
    return pl.pallas_call(
        _combine_body,
        out_shape=jax.ShapeDtypeStruct((B, QLEN, H, D), jnp.float32),
        in_specs=[
            pl.BlockSpec(memory_space=pltpu.VMEM),
            pl.BlockSpec(memory_space=pltpu.VMEM),
        ],
        out_specs=pl.BlockSpec(memory_space=pltpu.VMEM),
        scratch_shapes=[
            pltpu.VMEM((3, B, QLEN, H, D), jnp.bfloat16),
            pltpu.VMEM((3, B, QLEN, H, 1), jnp.float32),
            pltpu.SemaphoreType.DMA((2,)),
            pltpu.SemaphoreType.DMA((2,)),
            pltpu.SemaphoreType.DMA((2,)),
            pltpu.SemaphoreType.DMA((2,)),
        ],
        compiler_params=pltpu.CompilerParams(collective_id=0),
    )(o_part, l_part)


def kernel(Q, K, V):
    o_part, l_part = _local_flash(Q, K, V)
    return _combine(o_part, l_part)


# baseline (device time: 169013 ns/iter reference)
import jax
import jax.numpy as jnp
from jax import lax
from jax.experimental import pallas as pl
from jax.experimental.pallas import tpu as pltpu

B, QLEN, H, D = 8, 8, 16, 128
SCALE = D ** -0.5
N_DEV = 4


H_BLK = 8


def _flash_body(q_ref, k_ref, v_ref, o_ref, l_ref):
    for j in range(H_BLK):
        q = q_ref[0, :, j, :]
        k = k_ref[0, :, j, :]
        v = v_ref[0, :, j, :]
        s = lax.dot_general(
            q.astype(jnp.bfloat16),
            k.astype(jnp.bfloat16),
            (((1,), (1,)), ((), ())),
            preferred_element_type=jnp.float32,
        ) * SCALE
        p = jnp.exp(s)
        l = jnp.sum(p, axis=1, keepdims=True)
        o = lax.dot_general(
            p.astype(jnp.bfloat16),
            v.astype(jnp.bfloat16),
            (((1,), (0,)), ((), ())),
            preferred_element_type=jnp.float32,
        )
        o_ref[0, :, j, :] = o.astype(jnp.bfloat16)
        l_ref[0, :, j, :] = l


def _local_flash(Q, K, V):
    kv = K.shape[1]
    return pl.pallas_call(
        _flash_body,
        grid=(B, H // H_BLK),
        in_specs=[
            pl.BlockSpec((1, QLEN, H_BLK, D), lambda b, h: (b, 0, h, 0)),
            pl.BlockSpec((1, kv, H_BLK, D), lambda b, h: (b, 0, h, 0)),
            pl.BlockSpec((1, kv, H_BLK, D), lambda b, h: (b, 0, h, 0)),
        ],
        out_specs=[
            pl.BlockSpec((1, QLEN, H_BLK, D), lambda b, h: (b, 0, h, 0)),
            pl.BlockSpec((1, QLEN, H_BLK, 1), lambda b, h: (b, 0, h, 0)),
        ],
        out_shape=[
            jax.ShapeDtypeStruct((B, QLEN, H, D), jnp.bfloat16),
            jax.ShapeDtypeStruct((B, QLEN, H, 1), jnp.float32),
        ],
    )(Q, K, V)


def _combine_body(o_ref, l_ref, out_ref, co, cl, o_send, o_recv, l_send, l_recv):
    my = lax.axis_index("i")

    def exchange(o_src, l_src, partner, slot, sem):
        rdma_o = pltpu.make_async_remote_copy(
            src_ref=o_src,
            dst_ref=co.at[slot],
            send_sem=o_send.at[sem],
            recv_sem=o_recv.at[sem],
            device_id=(partner,),
            device_id_type=pl.DeviceIdType.MESH,
        )
        rdma_l = pltpu.make_async_remote_copy(
            src_ref=l_src,
            dst_ref=cl.at[slot],
            send_sem=l_send.at[sem],
            recv_sem=l_recv.at[sem],
            device_id=(partner,),
            device_id_type=pl.DeviceIdType.MESH,
        )
        rdma_o.start()
        rdma_l.start()
        rdma_o.wait()
        rdma_l.wait()

    exchange(o_ref, l_ref, my ^ 1, 0, 0)
    co[1] = (o_ref[...].astype(jnp.float32) + co[0].astype(jnp.float32)).astype(
        jnp.bfloat16
    )
    cl[1] = l_ref[...] + cl[0]

    exchange(co.at[1], cl.at[1], 3 - my, 2, 1)
    o_tot = co[1].astype(jnp.float32) + co[2].astype(jnp.float32)
    l_tot = cl[1] + cl[2]
    out_ref[...] = o_tot / l_tot


def _combine(o_part, l_part):
    return pl.pallas_call(
        _combine_body,
        out_shape=jax.ShapeDtypeStruct((B, QLEN, H, D), jnp.float32),
        in_specs=[
            pl.BlockSpec(memory_space=pltpu.VMEM),
            pl.BlockSpec(memory_space=pltpu.VMEM),
        ],
        out_specs=pl.BlockSpec(memory_space=pltpu.VMEM),
        scratch_shapes=[
            pltpu.VMEM((3, B, QLEN, H, D), jnp.bfloat16),
            pltpu.VMEM((3, B, QLEN, H, 1), jnp.float32),
            pltpu.SemaphoreType.DMA((2,)),
            pltpu.SemaphoreType.DMA((2,)),
            pltpu.SemaphoreType.DMA((2,)),
            pltpu.SemaphoreType.DMA((2,)),
        ],
    )(o_part, l_part)


def kernel(Q, K, V):
    o_part, l_part = _local_flash(Q, K, V)
    return _combine(o_part, l_part)


# device time: 77414 ns/iter; 2.1832x vs baseline; 2.1832x over previous
import jax
import jax.numpy as jnp
from jax import lax
from jax.experimental import pallas as pl
from jax.experimental.pallas import tpu as pltpu

B, QLEN, H, D = 8, 8, 16, 128
SCALE = D ** -0.5
N_DEV = 4


H_BLK = 8


def _flash_body(q_ref, k_ref, v_ref, o_ref, l_ref, mask_ref):
    kv = k_ref.shape[1]
    rows = kv * H_BLK
    cols = QLEN * H_BLK

    @pl.when((pl.program_id(0) == 0) & (pl.program_id(1) == 0))
    def _():
        r = lax.broadcasted_iota(jnp.int32, (rows, cols), 0)
        c = lax.broadcasted_iota(jnp.int32, (rows, cols), 1)
        mask_ref[...] = jnp.where((r % H_BLK) == (c % H_BLK), 1.0, 0.0).astype(
            jnp.float32
        )

    km = k_ref[0].reshape(rows, D)
    vm = v_ref[0].reshape(rows, D)
    qm = q_ref[0].reshape(cols, D) * SCALE

    s = lax.dot_general(
        km, qm, (((1,), (1,)), ((), ())), preferred_element_type=jnp.float32
    )
    p = jnp.exp(s) * mask_ref[...]

    o = lax.dot_general(
        p, vm, (((0,), (0,)), ((), ())), preferred_element_type=jnp.float32
    )
    l = lax.dot_general(
        p,
        jnp.ones((rows, 128), jnp.float32),
        (((0,), (0,)), ((), ())),
        preferred_element_type=jnp.float32,
    )
    o_ref[0] = o.reshape(QLEN, H_BLK, D).astype(jnp.bfloat16)
    l_ref[0] = l.reshape(QLEN, H_BLK, 128)[:, :, 0:1]


def _local_flash(Q, K, V):
    kv = K.shape[1]
    return pl.pallas_call(
        _flash_body,
        grid=(B, H // H_BLK),
        in_specs=[
            pl.BlockSpec((1, QLEN, H_BLK, D), lambda b, h: (b, 0, h, 0)),
            pl.BlockSpec((1, kv, H_BLK, D), lambda b, h: (b, 0, h, 0)),
            pl.BlockSpec((1, kv, H_BLK, D), lambda b, h: (b, 0, h, 0)),
        ],
        out_specs=[
            pl.BlockSpec((1, QLEN, H_BLK, D), lambda b, h: (b, 0, h, 0)),
            pl.BlockSpec((1, QLEN, H_BLK, 1), lambda b, h: (b, 0, h, 0)),
        ],
        out_shape=[
            jax.ShapeDtypeStruct((B, QLEN, H, D), jnp.bfloat16),
            jax.ShapeDtypeStruct((B, QLEN, H, 1), jnp.float32),
        ],
        scratch_shapes=[pltpu.VMEM((kv * H_BLK, QLEN * H_BLK), jnp.float32)],
    )(Q, K, V)


def _combine_body(o_ref, l_ref, out_ref, co, cl, o_send, o_recv, l_send, l_recv):
    my = lax.axis_index("i")

    def exchange(o_src, l_src, partner, slot, sem):
        rdma_o = pltpu.make_async_remote_copy(
            src_ref=o_src,
            dst_ref=co.at[slot],
            send_sem=o_send.at[sem],
            recv_sem=o_recv.at[sem],
            device_id=(partner,),
            device_id_type=pl.DeviceIdType.MESH,
        )
        rdma_l = pltpu.make_async_remote_copy(
            src_ref=l_src,
            dst_ref=cl.at[slot],
            send_sem=l_send.at[sem],
            recv_sem=l_recv.at[sem],
            device_id=(partner,),
            device_id_type=pl.DeviceIdType.MESH,
        )
        rdma_o.start()
        rdma_l.start()
        rdma_o.wait()
        rdma_l.wait()

    exchange(o_ref, l_ref, my ^ 1, 0, 0)
    co[1] = (o_ref[...].astype(jnp.float32) + co[0].astype(jnp.float32)).astype(
        jnp.bfloat16
    )
    cl[1] = l_ref[...] + cl[0]

    exchange(co.at[1], cl.at[1], 3 - my, 2, 1)
    o_tot = co[1].astype(jnp.float32) + co[2].astype(jnp.float32)
    l_tot = cl[1] + cl[2]
    out_ref[...] = o_tot / l_tot


def _combine(o_part, l_part):
    return pl.pallas_call(
        _combine_body,
        out_shape=jax.ShapeDtypeStruct((B, QLEN, H, D), jnp.float32),
        in_specs=[
            pl.BlockSpec(memory_space=pltpu.VMEM),
            pl.BlockSpec(memory_space=pltpu.VMEM),
        ],
        out_specs=pl.BlockSpec(memory_space=pltpu.VMEM),
        scratch_shapes=[
            pltpu.VMEM((3, B, QLEN, H, D), jnp.bfloat16),
            pltpu.VMEM((3, B, QLEN, H, 1), jnp.float32),
            pltpu.SemaphoreType.DMA((2,)),
            pltpu.SemaphoreType.DMA((2,)),
            pltpu.SemaphoreType.DMA((2,)),
            pltpu.SemaphoreType.DMA((2,)),
        ],
    )(o_part, l_part)


def kernel(Q, K, V):
    o_part, l_part = _local_flash(Q, K, V)
    return _combine(o_part, l_part)


# device time: 65127 ns/iter; 2.5951x vs baseline; 1.1887x over previous
import jax
import jax.numpy as jnp
from jax import lax
from jax.experimental import pallas as pl
from jax.experimental.pallas import tpu as pltpu

B, QLEN, H, D = 8, 8, 16, 128
SCALE = D ** -0.5
N_DEV = 4


H_BLK = 8


def _flash_body(q_ref, k_ref, v_ref, o_ref, l_ref, mask_ref):
    kv = k_ref.shape[1]
    rows = kv * H_BLK
    cols = QLEN * H_BLK

    @pl.when((pl.program_id(0) == 0) & (pl.program_id(1) == 0))
    def _():
        r = lax.broadcasted_iota(jnp.int32, (rows, cols), 0)
        c = lax.broadcasted_iota(jnp.int32, (rows, cols), 1)
        mask_ref[...] = jnp.where((r % H_BLK) == (c % H_BLK), 1.0, 0.0).astype(
            jnp.float32
        )

    km = k_ref[0].reshape(rows, D)
    vm = v_ref[0].reshape(rows, D)
    qm = q_ref[0].reshape(cols, D) * SCALE

    s = lax.dot_general(
        km, qm, (((1,), (1,)), ((), ())), preferred_element_type=jnp.float32
    )
    p = jnp.exp(s) * mask_ref[...]

    o = lax.dot_general(
        p, vm, (((0,), (0,)), ((), ())), preferred_element_type=jnp.float32
    )
    l = lax.dot_general(
        p,
        jnp.ones((rows, 128), jnp.float32),
        (((0,), (0,)), ((), ())),
        preferred_element_type=jnp.float32,
    )
    o_ref[0] = o.reshape(QLEN, H_BLK, D).astype(jnp.bfloat16)
    l_ref[0] = l.reshape(QLEN, H_BLK, 128)[:, :, 0].T


def _local_flash(Q, K, V):
    kv = K.shape[1]
    return pl.pallas_call(
        _flash_body,
        grid=(B, H // H_BLK),
        in_specs=[
            pl.BlockSpec((1, QLEN, H_BLK, D), lambda b, h: (b, 0, h, 0)),
            pl.BlockSpec((1, kv, H_BLK, D), lambda b, h: (b, 0, h, 0)),
            pl.BlockSpec((1, kv, H_BLK, D), lambda b, h: (b, 0, h, 0)),
        ],
        out_specs=[
            pl.BlockSpec((1, QLEN, H_BLK, D), lambda b, h: (b, 0, h, 0)),
            pl.BlockSpec((1, H_BLK, QLEN), lambda b, h: (b, h, 0)),
        ],
        out_shape=[
            jax.ShapeDtypeStruct((B, QLEN, H, D), jnp.bfloat16),
            jax.ShapeDtypeStruct((B, H, QLEN), jnp.float32),
        ],
        scratch_shapes=[pltpu.VMEM((kv * H_BLK, QLEN * H_BLK), jnp.float32)],
    )(Q, K, V)


def _combine_body(o_ref, l_ref, out_ref, co, cl, o_send, o_recv, l_send, l_recv):
    my = lax.axis_index("i")
    p1 = my ^ 1
    p2 = 3 - my

    barrier_sem = pltpu.get_barrier_semaphore()
    for nbr in (p1, p2):
        pl.semaphore_signal(
            barrier_sem, inc=1, device_id=(nbr,),
            device_id_type=pl.DeviceIdType.MESH,
        )
    pl.semaphore_wait(barrier_sem, 2)

    def exchange(o_src, l_src, partner, slot, sem):
        rdma_o = pltpu.make_async_remote_copy(
            src_ref=o_src,
            dst_ref=co.at[slot],
            send_sem=o_send.at[sem],
            recv_sem=o_recv.at[sem],
            device_id=(partner,),
            device_id_type=pl.DeviceIdType.MESH,
        )
        rdma_l = pltpu.make_async_remote_copy(
            src_ref=l_src,
            dst_ref=cl.at[slot],
            send_sem=l_send.at[sem],
            recv_sem=l_recv.at[sem],
            device_id=(partner,),
            device_id_type=pl.DeviceIdType.MESH,
        )
        rdma_o.start()
        rdma_l.start()
        return rdma_o, rdma_l

    o1, l1 = exchange(o_ref, l_ref, p1, 0, 0)
    o1.wait_recv()
    l1.wait_recv()
    co[1] = (o_ref[...].astype(jnp.float32) + co[0].astype(jnp.float32)).astype(
        jnp.bfloat16
    )
    cl[1] = l_ref[...] + cl[0]

    o2, l2 = exchange(co.at[1], cl.at[1], p2, 2, 1)
    o2.wait_recv()
    l2.wait_recv()
    o_tot = co[1].astype(jnp.float32) + co[2].astype(jnp.float32)
    l_tot = cl[1] + cl[2]
    out_ref[...] = o_tot / jnp.transpose(l_tot, (0, 2, 1))[..., None]
    o1.wait_send()
    l1.wait_send()
    o2.wait_send()
    l2.wait_send()


def _combine(o_part, l_part):
    return pl.pallas_call(
        _combine_body,
        out_shape=jax.ShapeDtypeStruct((B, QLEN, H, D), jnp.float32),
        in_specs=[
            pl.BlockSpec(memory_space=pltpu.VMEM),
            pl.BlockSpec(memory_space=pltpu.VMEM),
        ],
        out_specs=pl.BlockSpec(memory_space=pltpu.VMEM),
        scratch_shapes=[
            pltpu.VMEM((3, B, QLEN, H, D), jnp.bfloat16),
            pltpu.VMEM((3, B, H, QLEN), jnp.float32),
            pltpu.SemaphoreType.DMA((2,)),
            pltpu.SemaphoreType.DMA((2,)),
            pltpu.SemaphoreType.DMA((2,)),
            pltpu.SemaphoreType.DMA((2,)),
        ],
        compiler_params=pltpu.CompilerParams(collective_id=0),
    )(o_part, l_part)


def kernel(Q, K, V):
    o_part, l_part = _local_flash(Q, K, V)
    return _combine(o_part, l_part)


# device time: 59815 ns/iter; 2.8256x vs baseline; 1.0888x over previous
import jax
import jax.numpy as jnp
from jax import lax
from jax.experimental import pallas as pl
from jax.experimental.pallas import tpu as pltpu

B, QLEN, H, D = 8, 8, 16, 128
SCALE = D ** -0.5
H_BLK = 8
N_DEV = 4


def _body(
    q_ref, k_ref, v_ref, out_ref, mask_ref,
    o_mine, l_mine, co0, c1, co2, cl0, cl1, cl2,
    s1os, s1or, s2os, s2or, s1ls, s1lr, s2ls, s2lr,
):
    b = pl.program_id(0)
    hb = pl.program_id(1)
    my = lax.axis_index("i")
    p1 = my ^ 1
    p2 = 3 - my
    kv = k_ref.shape[1]
    rows = kv * H_BLK
    cols = QLEN * H_BLK

    @pl.when((b == 0) & (hb == 0))
    def _():
        barrier_sem = pltpu.get_barrier_semaphore()
        for nbr in (p1, p2):
            pl.semaphore_signal(
                barrier_sem, inc=1, device_id=(nbr,),
                device_id_type=pl.DeviceIdType.MESH,
            )
        pl.semaphore_wait(barrier_sem, 2)
        r = lax.broadcasted_iota(jnp.int32, (rows, cols), 0)
        c = lax.broadcasted_iota(jnp.int32, (rows, cols), 1)
        mask_ref[...] = jnp.where((r % H_BLK) == (c // QLEN), 1.0, 0.0).astype(
            jnp.float32
        )

    km = k_ref[0].reshape(rows, D)
    vm = v_ref[0].reshape(rows, D)
    qm = jnp.transpose(q_ref[0], (1, 0, 2)).reshape(cols, D) * SCALE
    s = lax.dot_general(
        km, qm, (((1,), (1,)), ((), ())), preferred_element_type=jnp.float32
    )
    p = jnp.exp(s) * mask_ref[...]
    o = lax.dot_general(
        p, vm, (((0,), (0,)), ((), ())), preferred_element_type=jnp.float32
    )
    l = lax.dot_general(
        p,
        jnp.ones((rows, 128), jnp.float32),
        (((0,), (0,)), ((), ())),
        preferred_element_type=jnp.float32,
    )
    o_mine[b, pl.ds(hb * H_BLK, H_BLK)] = o.reshape(H_BLK, QLEN, D).astype(
        jnp.bfloat16
    )
    l_mine[b, pl.ds(hb * H_BLK, H_BLK)] = l.reshape(H_BLK, QLEN, 128)[:, :, 0]

    def rdma(src, dst, ssem, rsem, partner):
        return pltpu.make_async_remote_copy(
            src_ref=src, dst_ref=dst, send_sem=ssem, recv_sem=rsem,
            device_id=(partner,), device_id_type=pl.DeviceIdType.MESH,
        )

    def stage1(i):
        return (
            rdma(o_mine.at[i], co0.at[i], s1os.at[i], s1or.at[i], p1),
            rdma(l_mine.at[i], cl0.at[i], s1ls.at[i], s1lr.at[i], p1),
        )

    def stage2(i):
        return (
            rdma(c1.at[i], co2.at[i], s2os.at[i], s2or.at[i], p2),
            rdma(cl1.at[i], cl2.at[i], s2ls.at[i], s2lr.at[i], p2),
        )

    def pair_sum_and_stage2(i):
        ro, rl = stage1(i)
        ro.wait_recv()
        rl.wait_recv()
        c1[i] = (o_mine[i].astype(jnp.float32) + co0[i].astype(jnp.float32)).astype(
            jnp.bfloat16
        )
        cl1[i] = l_mine[i] + cl0[i]
        so, sl = stage2(i)
        so.start()
        sl.start()

    def finalize_out(i):
        ro, rl = stage2(i)
        ro.wait_recv()
        rl.wait_recv()
        ot = c1[i].astype(jnp.float32) + co2[i].astype(jnp.float32)
        lt = cl1[i] + cl2[i]
        out_ref[i] = jnp.transpose(ot, (1, 0, 2)) / lt.T[..., None]

    @pl.when(hb == 1)
    def _():
        so, sl = stage1(b)
        so.start()
        sl.start()

    @pl.when((hb == 1) & (b >= 1))
    def _():
        pair_sum_and_stage2(b - 1)

    @pl.when((hb == 1) & (b >= 2))
    def _():
        finalize_out(b - 2)

    @pl.when((b == B - 1) & (hb == 1))
    def _():
        pair_sum_and_stage2(B - 1)
        finalize_out(B - 2)
        finalize_out(B - 1)
        for i in range(B):
            for d in stage1(i) + stage2(i):
                d.wait_send()


def kernel(Q, K, V):
    kv = K.shape[1]
    return pl.pallas_call(
        _body,
        grid=(B, H // H_BLK),
        in_specs=[
            pl.BlockSpec((1, QLEN, H_BLK, D), lambda b, h: (b, 0, h, 0)),
            pl.BlockSpec((1, kv, H_BLK, D), lambda b, h: (b, 0, h, 0)),
            pl.BlockSpec((1, kv, H_BLK, D), lambda b, h: (b, 0, h, 0)),
        ],
        out_specs=pl.BlockSpec((B, QLEN, H, D), lambda b, h: (0, 0, 0, 0)),
        out_shape=jax.ShapeDtypeStruct((B, QLEN, H, D), jnp.float32),
        scratch_shapes=[
            pltpu.VMEM((kv * H_BLK, QLEN * H_BLK), jnp.float32),
            pltpu.VMEM((B, H, QLEN, D), jnp.bfloat16),
            pltpu.VMEM((B, H, QLEN), jnp.float32),
            pltpu.VMEM((B, H, QLEN, D), jnp.bfloat16),
            pltpu.VMEM((B, H, QLEN, D), jnp.bfloat16),
            pltpu.VMEM((B, H, QLEN, D), jnp.bfloat16),
            pltpu.VMEM((B, H, QLEN), jnp.float32),
            pltpu.VMEM((B, H, QLEN), jnp.float32),
            pltpu.VMEM((B, H, QLEN), jnp.float32),
            pltpu.SemaphoreType.DMA((B,)),
            pltpu.SemaphoreType.DMA((B,)),
            pltpu.SemaphoreType.DMA((B,)),
            pltpu.SemaphoreType.DMA((B,)),
            pltpu.SemaphoreType.DMA((B,)),
            pltpu.SemaphoreType.DMA((B,)),
            pltpu.SemaphoreType.DMA((B,)),
            pltpu.SemaphoreType.DMA((B,)),
        ],
        compiler_params=pltpu.CompilerParams(collective_id=0),
    )(Q, K, V)
